# baseline (speedup 1.0000x reference)
import functools

import jax
import jax.numpy as jnp
from jax import lax
from jax.experimental import pallas as pl
from jax.experimental.pallas import tpu as pltpu
from jax.experimental.pallas import tpu_sc as plsc

N_TOKENS = 8192
D_MODEL = 4096
C_DIM = 256
N_CODES = 64
BN = 512


def _encode_block(x_ref, wenc_ref, benc_ref, g_ref, b_ref, embt_ref,
                  esq_ref, ze_ref, idx_ref, loss_ref):
    acc = jnp.dot(x_ref[...], wenc_ref[...],
                  preferred_element_type=jnp.float32) + benc_ref[...]
    mu = jnp.mean(acc, axis=-1, keepdims=True)
    var = jnp.mean((acc - mu) ** 2, axis=-1, keepdims=True)
    ze = (acc - mu) / jnp.sqrt(var + 1e-5) * g_ref[...] + b_ref[...]
    ze_ref[...] = ze
    zsq = jnp.sum(ze * ze, axis=-1, keepdims=True)
    cross = jnp.dot(ze, embt_ref[...], preferred_element_type=jnp.float32)
    d = zsq - 2.0 * cross + esq_ref[...]
    dmin = jnp.min(d, axis=1, keepdims=True)
    iota = lax.broadcasted_iota(jnp.int32, d.shape, 1)
    idx = jnp.min(jnp.where(d == dmin, iota, jnp.int32(2**30)), axis=1)
    idx_ref[...] = idx

    @pl.when(pl.program_id(0) == 0)
    def _():
        loss_ref[...] = jnp.zeros_like(loss_ref)

    loss_ref[...] += jnp.sum(dmin, axis=0, keepdims=True)


def _decode_table_block(emb_ref, wdec_ref, bdec_ref, out_ref):
    out_ref[...] = jnp.dot(emb_ref[...], wdec_ref[...],
                           preferred_element_type=jnp.float32) + bdec_ref[...]


def _recon_block(idx_ref, dec_ref, xr_ref):
    idx = idx_ref[...]
    onehot = (lax.broadcasted_iota(jnp.int32, (BN, N_CODES), 1)
              == idx[:, None]).astype(jnp.float32)
    xr_ref[...] = jnp.dot(onehot, dec_ref[...],
                          preferred_element_type=jnp.float32)


def _sc_info():
    try:
        info = plsc.get_sparse_core_info()
        return info.num_cores, info.num_subcores
    except Exception:
        return 2, 16


def _zq_gather_body(emb_hbm, idx_hbm, zq_hbm, idx_v, zq_v, sem,
                    *, n_cores, b_per_w):
    wid = lax.axis_index("s") * n_cores + lax.axis_index("c")
    base = wid * b_per_w
    pltpu.sync_copy(idx_hbm.at[pl.ds(base, b_per_w)], idx_v)
    rows_per_stream = 16
    n_sub = b_per_w // rows_per_stream
    descs = []
    for c in range(n_sub):
        lo = c * rows_per_stream
        sl = idx_v.at[pl.ds(lo, rows_per_stream)]
        descs.append(pltpu.async_copy(
            emb_hbm.at[sl], zq_v.at[pl.ds(lo, rows_per_stream)], sem))
    for d in descs:
        d.wait()
    pltpu.sync_copy(zq_v, zq_hbm.at[pl.ds(base, b_per_w)])


def kernel(x, modality, W_enc, b_enc, ln_g, ln_b, embeddings, W_dec, b_dec):
    del modality
    esq = jnp.sum(embeddings * embeddings, axis=-1).reshape(1, N_CODES)
    embt = embeddings.T

    n_blocks = N_TOKENS // BN
    ze, idx, loss_sum = pl.pallas_call(
        _encode_block,
        grid=(n_blocks,),
        in_specs=[
            pl.BlockSpec((BN, D_MODEL), lambda i: (i, 0)),
            pl.BlockSpec((D_MODEL, C_DIM), lambda i: (0, 0)),
            pl.BlockSpec((1, C_DIM), lambda i: (0, 0)),
            pl.BlockSpec((1, C_DIM), lambda i: (0, 0)),
            pl.BlockSpec((1, C_DIM), lambda i: (0, 0)),
            pl.BlockSpec((C_DIM, N_CODES), lambda i: (0, 0)),
            pl.BlockSpec((1, N_CODES), lambda i: (0, 0)),
        ],
        out_specs=[
            pl.BlockSpec((BN, C_DIM), lambda i: (i, 0)),
            pl.BlockSpec((BN,), lambda i: (i,)),
            pl.BlockSpec((1, 1), lambda i: (0, 0)),
        ],
        out_shape=[
            jax.ShapeDtypeStruct((N_TOKENS, C_DIM), jnp.float32),
            jax.ShapeDtypeStruct((N_TOKENS,), jnp.int32),
            jax.ShapeDtypeStruct((1, 1), jnp.float32),
        ],
        compiler_params=pltpu.CompilerParams(
            dimension_semantics=("arbitrary",)),
    )(x, W_enc, b_enc.reshape(1, C_DIM), ln_g.reshape(1, C_DIM),
      ln_b.reshape(1, C_DIM), embt, esq)

    decoded = pl.pallas_call(
        _decode_table_block,
        out_shape=jax.ShapeDtypeStruct((N_CODES, D_MODEL), jnp.float32),
    )(embeddings, W_dec, b_dec.reshape(1, D_MODEL))

    nc, ns = _sc_info()
    b_per_w = N_TOKENS // (nc * ns)
    mesh = plsc.VectorSubcoreMesh(core_axis_name="c", subcore_axis_name="s")
    z_q = pl.kernel(
        functools.partial(_zq_gather_body, n_cores=nc, b_per_w=b_per_w),
        out_type=jax.ShapeDtypeStruct((N_TOKENS, C_DIM), jnp.float32),
        mesh=mesh,
        scratch_types=[
            pltpu.VMEM((b_per_w,), jnp.int32),
            pltpu.VMEM((b_per_w, C_DIM), jnp.float32),
            pltpu.SemaphoreType.DMA,
        ],
    )(embeddings, idx)

    x_recon = pl.pallas_call(
        _recon_block,
        grid=(n_blocks,),
        in_specs=[
            pl.BlockSpec((BN,), lambda i: (i,)),
            pl.BlockSpec((N_CODES, D_MODEL), lambda i: (0, 0)),
        ],
        out_specs=pl.BlockSpec((BN, D_MODEL), lambda i: (i, 0)),
        out_shape=jax.ShapeDtypeStruct((N_TOKENS, D_MODEL), jnp.float32),
        compiler_params=pltpu.CompilerParams(
            dimension_semantics=("arbitrary",)),
    )(idx, decoded)

    loss = (loss_sum[0, 0] / (N_TOKENS * C_DIM)).reshape(())
    return (x_recon, loss, idx, ze, z_q)


# speedup vs baseline: 1.7816x; 1.7816x over previous
import functools

import jax
import jax.numpy as jnp
from jax import lax
from jax.experimental import pallas as pl
from jax.experimental.pallas import tpu as pltpu
from jax.experimental.pallas import tpu_sc as plsc

N_TOKENS = 8192
D_MODEL = 4096
C_DIM = 256
N_CODES = 64
BN = 512


def _encode_block(x_ref, wenc_ref, benc_ref, g_ref, b_ref, embt_ref,
                  esq_ref, ze_ref, idx_ref, loss_ref):
    acc = jnp.dot(x_ref[...], wenc_ref[...],
                  preferred_element_type=jnp.float32) + benc_ref[...]
    mu = jnp.mean(acc, axis=-1, keepdims=True)
    var = jnp.mean((acc - mu) ** 2, axis=-1, keepdims=True)
    ze = (acc - mu) / jnp.sqrt(var + 1e-5) * g_ref[...] + b_ref[...]
    ze_ref[...] = ze
    zsq = jnp.sum(ze * ze, axis=-1, keepdims=True)
    cross = jnp.dot(ze, embt_ref[...], preferred_element_type=jnp.float32)
    d = zsq - 2.0 * cross + esq_ref[...]
    dmin = jnp.min(d, axis=1, keepdims=True)
    iota = lax.broadcasted_iota(jnp.int32, d.shape, 1)
    idx = jnp.min(jnp.where(d == dmin, iota, jnp.int32(2**30)), axis=1)
    idx_ref[...] = idx

    @pl.when(pl.program_id(0) == 0)
    def _():
        loss_ref[...] = jnp.zeros_like(loss_ref)

    loss_ref[...] += jnp.sum(dmin, axis=0, keepdims=True)


def _decode_table_block(emb_ref, wdec_ref, bdec_ref, out_ref):
    out_ref[...] = jnp.dot(emb_ref[...], wdec_ref[...],
                           preferred_element_type=jnp.float32) + bdec_ref[...]


def _recon_block(idx_ref, dec_ref, emb_ref, xr_ref, zq_ref):
    idx = idx_ref[...]
    onehot = (lax.broadcasted_iota(jnp.int32, (BN, N_CODES), 1)
              == idx[:, None]).astype(jnp.float32)
    xr_ref[...] = jnp.dot(onehot, dec_ref[...],
                          preferred_element_type=jnp.float32)
    zq_ref[...] = jnp.dot(onehot, emb_ref[...],
                          preferred_element_type=jnp.float32,
                          precision=lax.Precision.HIGHEST)


def _sc_info():
    try:
        info = plsc.get_sparse_core_info()
        return info.num_cores, info.num_subcores
    except Exception:
        return 2, 16


def _zq_gather_body(emb_hbm, idx_hbm, zq_hbm, idx_v, zq_v, sem,
                    *, n_cores, b_per_w):
    wid = lax.axis_index("s") * n_cores + lax.axis_index("c")
    base = wid * b_per_w
    pltpu.sync_copy(idx_hbm.at[pl.ds(base, b_per_w)], idx_v)
    rows_per_stream = 16
    n_sub = b_per_w // rows_per_stream
    descs = []
    for c in range(n_sub):
        lo = c * rows_per_stream
        sl = idx_v.at[pl.ds(lo, rows_per_stream)]
        descs.append(pltpu.async_copy(
            emb_hbm.at[sl], zq_v.at[pl.ds(lo, rows_per_stream)], sem))
    for d in descs:
        d.wait()
    pltpu.sync_copy(zq_v, zq_hbm.at[pl.ds(base, b_per_w)])


def kernel(x, modality, W_enc, b_enc, ln_g, ln_b, embeddings, W_dec, b_dec):
    del modality
    esq = jnp.sum(embeddings * embeddings, axis=-1).reshape(1, N_CODES)
    embt = embeddings.T

    n_blocks = N_TOKENS // BN
    ze, idx, loss_sum = pl.pallas_call(
        _encode_block,
        grid=(n_blocks,),
        in_specs=[
            pl.BlockSpec((BN, D_MODEL), lambda i: (i, 0)),
            pl.BlockSpec((D_MODEL, C_DIM), lambda i: (0, 0)),
            pl.BlockSpec((1, C_DIM), lambda i: (0, 0)),
            pl.BlockSpec((1, C_DIM), lambda i: (0, 0)),
            pl.BlockSpec((1, C_DIM), lambda i: (0, 0)),
            pl.BlockSpec((C_DIM, N_CODES), lambda i: (0, 0)),
            pl.BlockSpec((1, N_CODES), lambda i: (0, 0)),
        ],
        out_specs=[
            pl.BlockSpec((BN, C_DIM), lambda i: (i, 0)),
            pl.BlockSpec((BN,), lambda i: (i,)),
            pl.BlockSpec((1, 1), lambda i: (0, 0)),
        ],
        out_shape=[
            jax.ShapeDtypeStruct((N_TOKENS, C_DIM), jnp.float32),
            jax.ShapeDtypeStruct((N_TOKENS,), jnp.int32),
            jax.ShapeDtypeStruct((1, 1), jnp.float32),
        ],
        compiler_params=pltpu.CompilerParams(
            dimension_semantics=("arbitrary",)),
    )(x, W_enc, b_enc.reshape(1, C_DIM), ln_g.reshape(1, C_DIM),
      ln_b.reshape(1, C_DIM), embt, esq)

    decoded = pl.pallas_call(
        _decode_table_block,
        out_shape=jax.ShapeDtypeStruct((N_CODES, D_MODEL), jnp.float32),
    )(embeddings, W_dec, b_dec.reshape(1, D_MODEL))

    x_recon, z_q = pl.pallas_call(
        _recon_block,
        grid=(n_blocks,),
        in_specs=[
            pl.BlockSpec((BN,), lambda i: (i,)),
            pl.BlockSpec((N_CODES, D_MODEL), lambda i: (0, 0)),
            pl.BlockSpec((N_CODES, C_DIM), lambda i: (0, 0)),
        ],
        out_specs=[
            pl.BlockSpec((BN, D_MODEL), lambda i: (i, 0)),
            pl.BlockSpec((BN, C_DIM), lambda i: (i, 0)),
        ],
        out_shape=[
            jax.ShapeDtypeStruct((N_TOKENS, D_MODEL), jnp.float32),
            jax.ShapeDtypeStruct((N_TOKENS, C_DIM), jnp.float32),
        ],
        compiler_params=pltpu.CompilerParams(
            dimension_semantics=("arbitrary",)),
    )(idx, decoded, embeddings)

    loss = (loss_sum[0, 0] / (N_TOKENS * C_DIM)).reshape(())
    return (x_recon, loss, idx, ze, z_q)
